# baseline (speedup 1.0000x reference)
import functools

import jax
import jax.numpy as jnp
from jax import lax
from jax.experimental import pallas as pl
from jax.experimental.pallas import tpu as pltpu
from jax.experimental.pallas import tpu_sc as plsc

N = 10000
E = 320000
D = 128
ALPHA = 0.5

NPAD = 10240
ROWS_PER_TILE = NPAD // 16
SUBC = 16
EPT = E // SUBC
CHUNK = 128
NCHUNK = 160
EPT_PAD = NCHUNK * CHUNK
NPAIR = NCHUNK // 2
NFULL = EPT // CHUNK
TAIL = EPT - NFULL * CHUNK


@functools.lru_cache(maxsize=1)
def _mesh():
    return plsc.VectorSubcoreMesh(core_axis_name="c", subcore_axis_name="s",
                                  num_cores=2, num_subcores=SUBC)


def _make_deg_kernel():
    def body(dstp, ones, zeros128, hist_out, dst2d, ones_v, hist_sh, sem, semi):
        c = lax.axis_index("c")
        s = lax.axis_index("s")
        rbase = s * ROWS_PER_TILE
        pltpu.sync_copy(ones, ones_v)
        pltpu.async_copy(dstp.at[c, s], dst2d, semi).wait()
        for j in range(ROWS_PER_TILE // 128):
            pltpu.sync_copy(zeros128, hist_sh.at[pl.ds(rbase + j * 128, 128)])
        plsc.subcore_barrier()

        def group(i, carry):
            for j in range(16):
                pltpu.async_copy(ones_v, hist_sh.at[dst2d.at[i * 16 + j]], sem,
                                 add=True)
            for j in range(16):
                pltpu.make_async_copy(zeros128, ones_v, sem).wait()
            return carry

        lax.fori_loop(0, NCHUNK // 16, group, 0)
        plsc.subcore_barrier()
        pltpu.sync_copy(hist_sh.at[pl.ds(rbase, ROWS_PER_TILE)],
                        hist_out.at[c, pl.ds(rbase, ROWS_PER_TILE)])

    return pl.kernel(
        body,
        out_type=jax.ShapeDtypeStruct((2, NPAD, D), jnp.float32),
        mesh=_mesh(),
        scratch_types=[
            pltpu.VMEM((NCHUNK, CHUNK), jnp.int32),
            pltpu.VMEM((CHUNK, D), jnp.float32),
            pltpu.VMEM_SHARED((NPAD, D), jnp.float32),
            pltpu.SemaphoreType.DMA,
            pltpu.SemaphoreType.DMA,
        ],
    )


NGRP = NCHUNK // 8
NSUPER = NGRP // 2


def _make_agg_kernel():
    def body(srcp, dstp, g_tbl, zeros128, acc_out, src_a, dst_a, src_b, dst_b,
             rows_a, rows_b, acc_sh, sem_si, sem_di, sem_ga, sem_gb, sem_s):
        c = lax.axis_index("c")
        s = lax.axis_index("s")
        rbase = s * ROWS_PER_TILE
        for j in range(ROWS_PER_TILE // 128):
            pltpu.sync_copy(zeros128, acc_sh.at[pl.ds(rbase + j * 128, 128)])
        plsc.subcore_barrier()

        gsrc = g_tbl.at[c]
        dummy_rows_hbm = g_tbl.at[0, pl.ds(0, CHUNK)]
        dummy_idx_hbm = srcp.at[0, 0, 0]
        rows = (rows_a, rows_b)
        gsems = (sem_ga, sem_gb)

        def idx_fetch(t, sbuf, dbuf):
            pltpu.async_copy(srcp.at[c, s, t], sbuf, sem_si)
            pltpu.async_copy(dstp.at[c, s, t], dbuf, sem_di)

        def src_drain(sbuf):
            pltpu.make_async_copy(dummy_idx_hbm, sbuf, sem_si).wait()

        def dst_drain(dbuf):
            pltpu.make_async_copy(dummy_idx_hbm, dbuf, sem_di).wait()

        def g_start(idx_row, b):
            pltpu.async_copy(gsrc.at[idx_row], rows[b], gsems[b])

        def g_wait(b):
            pltpu.make_async_copy(dummy_rows_hbm, rows[b], gsems[b]).wait()

        def s_start(idx_row, b):
            pltpu.async_copy(rows[b], acc_sh.at[idx_row], sem_s, add=True)

        def s_drain():
            pltpu.make_async_copy(dummy_rows_hbm, rows_a, sem_s).wait()

        def process_group(sbuf, dbuf, nsbuf, has_next, needs_dst_drain,
                          first_ever):
            @pl.when(needs_dst_drain)
            def _():
                dst_drain(dbuf)

            for j in range(8):
                if j < 7:
                    if j == 0:
                        @pl.when(jnp.logical_not(first_ever))
                        def _():
                            s_drain()
                    else:
                        s_drain()
                    g_start(sbuf.at[j + 1], (j + 1) % 2)
                else:
                    @pl.when(has_next)
                    def _():
                        src_drain(nsbuf)
                        s_drain()
                        g_start(nsbuf.at[0], (j + 1) % 2)
                g_wait(j % 2)
                s_start(dbuf.at[j], j % 2)

        pltpu.async_copy(srcp.at[c, s, 0], src_a, sem_si).wait()
        pltpu.async_copy(dstp.at[c, s, 0], dst_a, sem_di).wait()
        g_start(src_a.at[0], 0)

        def super_body(i, carry):
            t = 2 * i
            idx_fetch(t + 1, src_b, dst_b)
            process_group(src_a, dst_a, src_b, has_next=(i >= 0),
                          needs_dst_drain=(i > 0), first_ever=(i == 0))

            @pl.when(i < NSUPER - 1)
            def _():
                idx_fetch(t + 2, src_a, dst_a)

            process_group(src_b, dst_b, src_a, has_next=(i < NSUPER - 1),
                          needs_dst_drain=(i >= 0),
                          first_ever=jnp.bool_(False))
            return carry

        lax.fori_loop(0, NSUPER, super_body, 0)
        s_drain()
        s_drain()
        plsc.subcore_barrier()
        pltpu.sync_copy(acc_sh.at[pl.ds(rbase, ROWS_PER_TILE)],
                        acc_out.at[c, pl.ds(rbase, ROWS_PER_TILE)])

    return pl.kernel(
        body,
        out_type=jax.ShapeDtypeStruct((2, NPAD, D), jnp.float32),
        mesh=_mesh(),
        scratch_types=[
            pltpu.VMEM((8, CHUNK), jnp.int32),
            pltpu.VMEM((8, CHUNK), jnp.int32),
            pltpu.VMEM((8, CHUNK), jnp.int32),
            pltpu.VMEM((8, CHUNK), jnp.int32),
            pltpu.VMEM((CHUNK, D), jnp.float32),
            pltpu.VMEM((CHUNK, D), jnp.float32),
            pltpu.VMEM_SHARED((NPAD, D), jnp.float32),
            pltpu.SemaphoreType.DMA,
            pltpu.SemaphoreType.DMA,
            pltpu.SemaphoreType.DMA,
            pltpu.SemaphoreType.DMA,
            pltpu.SemaphoreType.DMA,
        ],
    )


_deg_kernel_c = functools.lru_cache(maxsize=1)(_make_deg_kernel)
_agg_kernel_c = functools.lru_cache(maxsize=1)(_make_agg_kernel)

_BROWS = 1000


def _scale_matmul_body(x_ref, w_ref, hist_ref, g_ref):
    g = pl.program_id(0)
    h = jnp.dot(x_ref[...], w_ref[0].T, preferred_element_type=jnp.float32)
    deg = hist_ref[0, :, 0:1]
    inv = jnp.where(deg > 0, lax.rsqrt(deg), 0.0)
    scale = jnp.where(g == 0, ALPHA, 1.0 - ALPHA)
    g_ref[0] = (scale * inv) * h


def _combine_body(acc_ref, hist_ref, b1_ref, b2_ref, out_ref):
    d0 = hist_ref[0, :, 0:1]
    d1 = hist_ref[1, :, 0:1]
    inv0 = jnp.where(d0 > 0, lax.rsqrt(d0), 0.0)
    inv1 = jnp.where(d1 > 0, lax.rsqrt(d1), 0.0)
    bias = ALPHA * b1_ref[0] + (1.0 - ALPHA) * b2_ref[0]
    out_ref[...] = inv0 * acc_ref[0] + inv1 * acc_ref[1] + bias[None, :]


@jax.jit
def kernel(x, edge_index, W1, b1, W2, b2):
    ones128 = jnp.ones((CHUNK, D), jnp.float32)
    zeros128 = jnp.zeros((128, D), jnp.float32)

    src = edge_index[::-1].reshape(2, SUBC, EPT)
    dst = edge_index.reshape(2, SUBC, EPT)
    srcp = jnp.pad(src, ((0, 0), (0, 0), (0, EPT_PAD - EPT)),
                   constant_values=0).reshape(2, SUBC, NCHUNK, CHUNK)
    dstp = jnp.pad(dst, ((0, 0), (0, 0), (0, EPT_PAD - EPT)),
                   constant_values=NPAD - 1).reshape(2, SUBC, NCHUNK, CHUNK)

    hist = _deg_kernel_c()(dstp, ones128, zeros128)

    wstack = jnp.stack([W1, W2])
    g_tbl = pl.pallas_call(
        _scale_matmul_body,
        grid=(2, N // _BROWS),
        in_specs=[
            pl.BlockSpec((_BROWS, D), lambda g, i: (i, 0)),
            pl.BlockSpec((1, D, D), lambda g, i: (g, 0, 0)),
            pl.BlockSpec((1, _BROWS, D), lambda g, i: (1 - g, i, 0)),
        ],
        out_specs=pl.BlockSpec((1, _BROWS, D), lambda g, i: (g, i, 0)),
        out_shape=jax.ShapeDtypeStruct((2, N, D), jnp.float32),
    )(x, wstack, hist)

    srcp5 = srcp.reshape(2, SUBC, NGRP, 8, CHUNK)
    dstp5 = dstp.reshape(2, SUBC, NGRP, 8, CHUNK)
    acc = _agg_kernel_c()(srcp5, dstp5, g_tbl, zeros128)

    out = pl.pallas_call(
        _combine_body,
        grid=(N // _BROWS,),
        in_specs=[
            pl.BlockSpec((2, _BROWS, D), lambda i: (0, i, 0)),
            pl.BlockSpec((2, _BROWS, D), lambda i: (0, i, 0)),
            pl.BlockSpec((1, D), lambda i: (0, 0)),
            pl.BlockSpec((1, D), lambda i: (0, 0)),
        ],
        out_specs=pl.BlockSpec((_BROWS, D), lambda i: (i, 0)),
        out_shape=jax.ShapeDtypeStruct((N, D), jnp.float32),
    )(acc, hist, b1.reshape(1, D), b2.reshape(1, D))
    return out


# speedup vs baseline: 1.0078x; 1.0078x over previous
import functools

import jax
import jax.numpy as jnp
from jax import lax
from jax.experimental import pallas as pl
from jax.experimental.pallas import tpu as pltpu
from jax.experimental.pallas import tpu_sc as plsc

N = 10000
E = 320000
D = 128
ALPHA = 0.5

NPAD = 10240
ROWS_PER_TILE = NPAD // 16
SUBC = 16
EPT = E // SUBC
CHUNK = 128
NCHUNK = 160
EPT_PAD = NCHUNK * CHUNK
NPAIR = NCHUNK // 2
NFULL = EPT // CHUNK
TAIL = EPT - NFULL * CHUNK


@functools.lru_cache(maxsize=1)
def _mesh():
    return plsc.VectorSubcoreMesh(core_axis_name="c", subcore_axis_name="s",
                                  num_cores=2, num_subcores=SUBC)


def _make_deg_kernel():
    def body(dstp, ones, zeros128, hist_out, dst2d, ones_v, hist_sh, sem, semi):
        c = lax.axis_index("c")
        s = lax.axis_index("s")
        rbase = s * ROWS_PER_TILE
        pltpu.sync_copy(ones, ones_v)
        pltpu.async_copy(dstp.at[c, s], dst2d, semi).wait()
        for j in range(ROWS_PER_TILE // 128):
            pltpu.sync_copy(zeros128, hist_sh.at[pl.ds(rbase + j * 128, 128)])
        plsc.subcore_barrier()

        def group(i, carry):
            for j in range(16):
                pltpu.async_copy(ones_v, hist_sh.at[dst2d.at[i * 16 + j]], sem,
                                 add=True)
            for j in range(16):
                pltpu.make_async_copy(zeros128, ones_v, sem).wait()
            return carry

        lax.fori_loop(0, NCHUNK // 16, group, 0)
        plsc.subcore_barrier()
        pltpu.sync_copy(hist_sh.at[pl.ds(rbase, ROWS_PER_TILE)],
                        hist_out.at[c, pl.ds(rbase, ROWS_PER_TILE)])

    return pl.kernel(
        body,
        out_type=jax.ShapeDtypeStruct((2, NPAD, D), jnp.float32),
        mesh=_mesh(),
        scratch_types=[
            pltpu.VMEM((NCHUNK, CHUNK), jnp.int32),
            pltpu.VMEM((CHUNK, D), jnp.float32),
            pltpu.VMEM_SHARED((NPAD, D), jnp.float32),
            pltpu.SemaphoreType.DMA,
            pltpu.SemaphoreType.DMA,
        ],
    )


def _make_agg_kernel():
    def body(ei, g_tbl, zeros128, acc_out, idx_s, idx_d, idx_st, idx_dt,
             rows, rows_t, acc_sh, sem):
        c = lax.axis_index("c")
        s = lax.axis_index("s")
        rbase = s * ROWS_PER_TILE
        for j in range(ROWS_PER_TILE // 128):
            pltpu.sync_copy(zeros128, acc_sh.at[pl.ds(rbase + j * 128, 128)])
        plsc.subcore_barrier()

        def chunk(si_ref, di_ref, rows_ref, off):
            n = si_ref.shape[0]
            pltpu.async_copy(ei.at[pl.ds((1 - c) * E + off, n)], si_ref, sem).wait()
            pltpu.async_copy(ei.at[pl.ds(c * E + off, n)], di_ref, sem).wait()
            pltpu.async_copy(g_tbl.at[c].at[si_ref], rows_ref, sem).wait()
            pltpu.sync_copy(rows_ref, acc_sh.at[di_ref], add=True)

        def loop_body(k, carry):
            chunk(idx_s, idx_d, rows, s * EPT + k * CHUNK)
            return carry

        lax.fori_loop(0, NFULL, loop_body, 0)
        chunk(idx_st, idx_dt, rows_t, s * EPT + NFULL * CHUNK)
        plsc.subcore_barrier()
        pltpu.sync_copy(acc_sh.at[pl.ds(rbase, ROWS_PER_TILE)],
                        acc_out.at[c, pl.ds(rbase, ROWS_PER_TILE)])

    return pl.kernel(
        body,
        out_type=jax.ShapeDtypeStruct((2, NPAD, D), jnp.float32),
        mesh=_mesh(),
        scratch_types=[
            pltpu.VMEM((CHUNK,), jnp.int32),
            pltpu.VMEM((CHUNK,), jnp.int32),
            pltpu.VMEM((TAIL,), jnp.int32),
            pltpu.VMEM((TAIL,), jnp.int32),
            pltpu.VMEM((CHUNK, D), jnp.float32),
            pltpu.VMEM((TAIL, D), jnp.float32),
            pltpu.VMEM_SHARED((NPAD, D), jnp.float32),
            pltpu.SemaphoreType.DMA,
        ],
    )


_deg_kernel_c = functools.lru_cache(maxsize=1)(_make_deg_kernel)
_agg_kernel_c = functools.lru_cache(maxsize=1)(_make_agg_kernel)

_BROWS = 1000


def _scale_matmul_body(x_ref, w_ref, hist_ref, g_ref):
    g = pl.program_id(0)
    h = jnp.dot(x_ref[...], w_ref[0].T, preferred_element_type=jnp.float32)
    deg = hist_ref[0, :, 0:1]
    inv = jnp.where(deg > 0, lax.rsqrt(deg), 0.0)
    scale = jnp.where(g == 0, ALPHA, 1.0 - ALPHA)
    g_ref[0] = (scale * inv) * h


def _combine_body(acc_ref, hist_ref, b1_ref, b2_ref, out_ref):
    d0 = hist_ref[0, :, 0:1]
    d1 = hist_ref[1, :, 0:1]
    inv0 = jnp.where(d0 > 0, lax.rsqrt(d0), 0.0)
    inv1 = jnp.where(d1 > 0, lax.rsqrt(d1), 0.0)
    bias = ALPHA * b1_ref[0] + (1.0 - ALPHA) * b2_ref[0]
    out_ref[...] = inv0 * acc_ref[0] + inv1 * acc_ref[1] + bias[None, :]


@jax.jit
def kernel(x, edge_index, W1, b1, W2, b2):
    ones128 = jnp.ones((CHUNK, D), jnp.float32)
    zeros128 = jnp.zeros((128, D), jnp.float32)

    src = edge_index[::-1].reshape(2, SUBC, EPT)
    dst = edge_index.reshape(2, SUBC, EPT)
    srcp = jnp.pad(src, ((0, 0), (0, 0), (0, EPT_PAD - EPT)),
                   constant_values=0).reshape(2, SUBC, NCHUNK, CHUNK)
    dstp = jnp.pad(dst, ((0, 0), (0, 0), (0, EPT_PAD - EPT)),
                   constant_values=NPAD - 1).reshape(2, SUBC, NCHUNK, CHUNK)

    hist = _deg_kernel_c()(dstp, ones128, zeros128)

    wstack = jnp.stack([W1, W2])
    g_tbl = pl.pallas_call(
        _scale_matmul_body,
        grid=(2, N // _BROWS),
        in_specs=[
            pl.BlockSpec((_BROWS, D), lambda g, i: (i, 0)),
            pl.BlockSpec((1, D, D), lambda g, i: (g, 0, 0)),
            pl.BlockSpec((1, _BROWS, D), lambda g, i: (1 - g, i, 0)),
        ],
        out_specs=pl.BlockSpec((1, _BROWS, D), lambda g, i: (g, i, 0)),
        out_shape=jax.ShapeDtypeStruct((2, N, D), jnp.float32),
    )(x, wstack, hist)

    ei_flat = edge_index.reshape(-1)
    acc = _agg_kernel_c()(ei_flat, g_tbl, zeros128)

    out = pl.pallas_call(
        _combine_body,
        grid=(N // _BROWS,),
        in_specs=[
            pl.BlockSpec((2, _BROWS, D), lambda i: (0, i, 0)),
            pl.BlockSpec((2, _BROWS, D), lambda i: (0, i, 0)),
            pl.BlockSpec((1, D), lambda i: (0, 0)),
            pl.BlockSpec((1, D), lambda i: (0, 0)),
        ],
        out_specs=pl.BlockSpec((_BROWS, D), lambda i: (i, 0)),
        out_shape=jax.ShapeDtypeStruct((N, D), jnp.float32),
    )(acc, hist, b1.reshape(1, D), b2.reshape(1, D))
    return out
